# baseline (speedup 1.0000x reference)
import functools

import jax
import jax.numpy as jnp
from jax import lax
from jax.experimental import pallas as pl
from jax.experimental.pallas import tpu as pltpu
from jax.experimental.pallas import tpu_sc as plsc

_D = 64
_NC = 2
_NS = 16
_NW = _NC * _NS
_CH = 128


@functools.lru_cache(maxsize=None)
def _make_emb(B: int, V: int):
    n_ch_total = B // _CH
    n_ch = n_ch_total // _NW
    assert n_ch * _NW * _CH == B

    mesh = plsc.VectorSubcoreMesh(
        core_axis_name="c", subcore_axis_name="s",
        num_cores=_NC, num_subcores=_NS,
    )

    @functools.partial(
        pl.kernel,
        mesh=mesh,
        compiler_params=pltpu.CompilerParams(use_tc_tiling_on_sc=False),
        out_type=jax.ShapeDtypeStruct((B, _D), jnp.float32),
        scratch_types=[
            pltpu.VMEM((n_ch, _CH), jnp.int32),
            pltpu.VMEM((2, _CH, _D), jnp.float32),
            pltpu.SemaphoreType.DMA,
            pltpu.SemaphoreType.DMA,
        ],
    )
    def emb(idx_hbm, table_hbm, out_hbm, idx_v, rows_v, g_sem, o_sem):
        wid = lax.axis_index("s") * _NC + lax.axis_index("c")
        first = wid * n_ch
        base = first * _CH
        pltpu.sync_copy(idx_hbm.at[pl.ds(first, n_ch)], idx_v)

        def gather(j):
            return pltpu.async_copy(
                table_hbm.at[idx_v.at[j]], rows_v.at[j % 2], g_sem)

        def out_copy(j):
            return pltpu.async_copy(
                rows_v.at[j % 2], out_hbm.at[pl.ds(base + j * _CH, _CH)], o_sem)

        o_copies = [None] * n_ch
        g_cur = gather(0)
        for j in range(n_ch):
            g_cur.wait()
            if j + 1 < n_ch:
                if j >= 1:
                    o_copies[j - 1].wait()
                g_cur = gather(j + 1)
            o_copies[j] = out_copy(j)
        if n_ch >= 2:
            o_copies[n_ch - 2].wait()
        o_copies[n_ch - 1].wait()

    return emb


def kernel(tokens, table):
    orig_shape = tokens.shape
    B = 1
    for s in orig_shape:
        B *= s
    idx = tokens.reshape(B // _CH, _CH).astype(jnp.int32)
    out = _make_emb(B, table.shape[0])(idx, table)
    return out.reshape(*orig_shape, _D)


# speedup vs baseline: 1.0620x; 1.0620x over previous
import functools

import jax
import jax.numpy as jnp
from jax import lax
from jax.experimental import pallas as pl
from jax.experimental.pallas import tpu as pltpu
from jax.experimental.pallas import tpu_sc as plsc

_D = 64
_NC = 2
_NS = 16
_NW = _NC * _NS
_CH = 128
_NB = 8


@functools.lru_cache(maxsize=None)
def _make_emb(B: int, V: int):
    n_ch_total = B // _CH
    n_ch = n_ch_total // _NW
    assert n_ch * _NW * _CH == B

    mesh = plsc.VectorSubcoreMesh(
        core_axis_name="c", subcore_axis_name="s",
        num_cores=_NC, num_subcores=_NS,
    )

    @functools.partial(
        pl.kernel,
        mesh=mesh,
        compiler_params=pltpu.CompilerParams(use_tc_tiling_on_sc=False),
        out_type=jax.ShapeDtypeStruct((B, _D), jnp.float32),
        scratch_types=[
            pltpu.VMEM((n_ch, _CH), jnp.int32),
            pltpu.VMEM((_NB, _CH, _D), jnp.float32),
            pltpu.SemaphoreType.DMA,
            pltpu.SemaphoreType.DMA,
        ],
    )
    def emb(idx_hbm, table_hbm, out_hbm, idx_v, rows_v, g_sem, o_sem):
        wid = lax.axis_index("s") * _NC + lax.axis_index("c")
        first = wid * n_ch
        base = first * _CH
        pltpu.sync_copy(idx_hbm.at[pl.ds(first, n_ch)], idx_v)

        def gather(j):
            return pltpu.async_copy(
                table_hbm.at[idx_v.at[j]], rows_v.at[j % _NB], g_sem)

        def out_copy(j):
            return pltpu.async_copy(
                rows_v.at[j % _NB],
                out_hbm.at[pl.ds(base + j * _CH, _CH)], o_sem)

        gathers = [gather(j) for j in range(min(_NB, n_ch))]
        o_copies = [None] * n_ch
        for j in range(n_ch):
            gathers[j].wait()
            o_copies[j] = out_copy(j)
            if j + _NB < n_ch:
                o_copies[j].wait()
                gathers.append(gather(j + _NB))
        for j in range(max(0, n_ch - _NB), n_ch):
            if o_copies[j] is not None and j + _NB >= n_ch:
                o_copies[j].wait()

    return emb


def kernel(tokens, table):
    orig_shape = tokens.shape
    B = 1
    for s in orig_shape:
        B *= s
    idx = tokens.reshape(B // _CH, _CH).astype(jnp.int32)
    out = _make_emb(B, table.shape[0])(idx, table)
    return out.reshape(*orig_shape, _D)


# speedup vs baseline: 1.4995x; 1.4120x over previous
import functools

import jax
import jax.numpy as jnp
from jax import lax
from jax.experimental import pallas as pl
from jax.experimental.pallas import tpu as pltpu
from jax.experimental.pallas import tpu_sc as plsc

_D = 64
_DP = 128
_NC = 2
_NS = 16
_NW = _NC * _NS
_CH = 128
_NB = 8
_L = 16


@functools.lru_cache(maxsize=None)
def _make_emb(B: int, V: int):
    n_ch_total = B // _CH
    n_ch = n_ch_total // _NW
    assert n_ch * _NW * _CH == B

    mesh = plsc.VectorSubcoreMesh(
        core_axis_name="c", subcore_axis_name="s",
        num_cores=_NC, num_subcores=_NS,
    )

    @functools.partial(
        pl.kernel,
        mesh=mesh,
        compiler_params=pltpu.CompilerParams(use_tc_tiling_on_sc=False),
        out_type=jax.ShapeDtypeStruct((B, _DP), jnp.float32),
        scratch_types=[
            pltpu.VMEM((n_ch, _CH), jnp.int32),
            pltpu.VMEM((n_ch, _CH), jnp.int32),
            pltpu.VMEM((_NB, _CH, _D), jnp.float32),
            pltpu.SemaphoreType.DMA,
            pltpu.SemaphoreType.DMA,
        ],
    )
    def emb(idx_hbm, table_hbm, out_hbm, idx_v, idx2_v, rows_v, g_sem, o_sem):
        wid = lax.axis_index("s") * _NC + lax.axis_index("c")
        first = wid * n_ch
        base = first * _CH
        pltpu.sync_copy(idx_hbm.at[pl.ds(first, n_ch)], idx_v)
        for j in range(n_ch):
            for k in range(_CH // _L):
                idx2_v[j, pl.ds(k * _L, _L)] = idx_v[j, pl.ds(k * _L, _L)] * 2

        def gather(j):
            return pltpu.async_copy(
                table_hbm.at[idx2_v.at[j]], rows_v.at[j % _NB], g_sem)

        def out_copy(j):
            return pltpu.async_copy(
                rows_v.at[j % _NB],
                out_hbm.at[pl.ds(base + j * _CH, _CH), pl.ds(0, _D)], o_sem)

        gathers = [gather(j) for j in range(min(_NB, n_ch))]
        o_copies = [None] * n_ch
        for j in range(n_ch):
            gathers[j].wait()
            o_copies[j] = out_copy(j)
            if j + _NB < n_ch:
                o_copies[j].wait()
                gathers.append(gather(j + _NB))
        for j in range(max(0, n_ch - _NB), n_ch):
            o_copies[j].wait()

    return emb


def kernel(tokens, table):
    orig_shape = tokens.shape
    B = 1
    for s in orig_shape:
        B *= s
    V = table.shape[0]
    idx = tokens.reshape(B // _CH, _CH).astype(jnp.int32)
    tab128 = jnp.pad(table, ((0, 0), (0, _DP - _D)))
    tab2 = tab128.reshape(2 * V, _D)
    out = _make_emb(B, V)(idx, tab2)
    return out[:, :_D].reshape(*orig_shape, _D)
